# Initial kernel scaffold; baseline (speedup 1.0000x reference)
import jax
import jax.numpy as jnp
from jax.experimental import pallas as pl


def kernel(input_ids, table, gamma, beta):
    raise NotImplementedError("write your pallas kernel here")


# speedup vs baseline: 1.6982x; 1.6982x over previous
import functools

import jax
import jax.numpy as jnp
from jax import lax
from jax.experimental import pallas as pl
from jax.experimental.pallas import tpu as pltpu
from jax.experimental.pallas import tpu_sc as plsc

_DIM = 64
_EPS = 1e-05

_NC = 2
_NS = 16
_NW = _NC * _NS

_CHUNK = 128
_NBUF = 4


@functools.lru_cache(maxsize=None)
def _make_sc_kernel(total, vocab):
    per_w = total // _NW
    nch = per_w // _CHUNK
    assert per_w * _NW == total and nch * _CHUNK == per_w and nch % _NBUF == 0

    mesh = plsc.VectorSubcoreMesh(core_axis_name="c", subcore_axis_name="s")
    scratch = (
        [pltpu.VMEM((nch, _CHUNK), jnp.int32)]
        + [pltpu.VMEM((_CHUNK, _DIM), jnp.float32) for _ in range(2 * _NBUF)]
        + [pltpu.VMEM((_DIM,), jnp.float32) for _ in range(2)]
        + [pltpu.SemaphoreType.DMA for _ in range(2 * _NBUF)]
    )

    @functools.partial(
        pl.kernel,
        mesh=mesh,
        out_type=jax.ShapeDtypeStruct((total, _DIM), jnp.float32),
        scratch_types=scratch,
        compiler_params=pltpu.CompilerParams(use_tc_tiling_on_sc=False),
    )
    def body(idx_hbm, table_hbm, gamma_hbm, beta_hbm, out_hbm, idx_v, *rest):
        gbufs = rest[0:_NBUF]
        obufs = rest[_NBUF:2 * _NBUF]
        g_v = rest[2 * _NBUF]
        b_v = rest[2 * _NBUF + 1]
        gsem = rest[2 * _NBUF + 2: 3 * _NBUF + 2]
        ssem = rest[3 * _NBUF + 2: 4 * _NBUF + 2]

        wid = lax.axis_index("s") * _NC + lax.axis_index("c")
        base = wid * per_w
        pltpu.sync_copy(idx_hbm.at[wid], idx_v)
        pltpu.sync_copy(gamma_hbm, g_v)
        pltpu.sync_copy(beta_hbm, b_v)

        gvec = [g_v[pl.ds(16 * j, 16)] for j in range(4)]
        bvec = [b_v[pl.ds(16 * j, 16)] for j in range(4)]

        lane = lax.iota(jnp.int32, 16)
        perms = [(lane ^ d)[:, None] for d in (8, 4, 2, 1)]
        _dnums = lax.GatherDimensionNumbers(
            offset_dims=(), collapsed_slice_dims=(0,), start_index_map=(0,))

        def lane_sum(v):
            for p in perms:
                v = v + lax.gather(v, p, _dnums, (1,),
                                   mode=lax.GatherScatterMode.PROMISE_IN_BOUNDS)
            return v

        def gather_copy(ch, b):
            return pltpu.make_async_copy(
                table_hbm.at[idx_v.at[ch]], gbufs[b], gsem[b])

        def store_copy(ch, b):
            return pltpu.make_async_copy(
                obufs[b], out_hbm.at[pl.ds(base + ch * _CHUNK, _CHUNK)], ssem[b])

        for b in range(_NBUF):
            gather_copy(b, b).start()

        def compute(ch, b):
            gb = gbufs[b]
            ob = obufs[b]

            def grp16(gi, carry):
                ivec = idx_v[ch, pl.ds(gi * 16, 16)]
                mvec = jnp.where(ivec == 0, 0.0, 1.0).astype(jnp.float32)
                for k in range(16):
                    r = gi * 16 + k
                    m = mvec[k]
                    xs = [gb[r, pl.ds(16 * j, 16)] * m for j in range(4)]
                    s = (xs[0] + xs[1]) + (xs[2] + xs[3])
                    q = (xs[0] * xs[0] + xs[1] * xs[1]) + (xs[2] * xs[2] + xs[3] * xs[3])
                    mean = lane_sum(s) * (1.0 / 64.0)
                    var = jnp.maximum(
                        lane_sum(q) * (1.0 / 64.0) - mean * mean, 0.0) + _EPS
                    i0 = lax.bitcast_convert_type(var, jnp.int32)
                    y = lax.bitcast_convert_type(
                        jnp.int32(0x5F3759DF) - lax.shift_right_arithmetic(i0, 1),
                        jnp.float32)
                    hv = 0.5 * var
                    y = y * (1.5 - hv * y * y)
                    y = y * (1.5 - hv * y * y)
                    y = y * (1.5 - hv * y * y)
                    for j in range(4):
                        ob[r, pl.ds(16 * j, 16)] = (xs[j] - mean) * y * gvec[j] + bvec[j]
                return carry

            lax.fori_loop(0, _CHUNK // 16, grp16, 0)

        def grp(i, carry):
            for b in range(_NBUF):
                ch = i * _NBUF + b
                gather_copy(ch, b).wait()

                @pl.when(ch >= _NBUF)
                def _():
                    store_copy(ch - _NBUF, b).wait()

                compute(ch, b)

                @pl.when(ch + _NBUF < nch)
                def _():
                    gather_copy(ch + _NBUF, b).start()

                store_copy(ch, b).start()
            return carry

        lax.fori_loop(0, nch // _NBUF, grp, 0)

        for b in range(_NBUF):
            store_copy(nch - _NBUF + b, b).wait()

    return body


def kernel(input_ids, table, gamma, beta):
    batch, hist = input_ids.shape
    total = batch * hist
    idx = input_ids.astype(jnp.int32).reshape(_NW, total // (_NW * _CHUNK), _CHUNK)
    sc = _make_sc_kernel(total, table.shape[0])
    out = sc(idx, table, gamma, beta)
    return out.reshape(batch, hist, _DIM)
